# baseline (speedup 1.0000x reference)
import jax
import jax.numpy as jnp
from jax import lax
from jax.experimental import pallas as pl
from jax.experimental.pallas import tpu as pltpu

_DN_T = (((1,), (1,)), ((), ()))


def _body(adj_ref, x_ref, w0_ref, w1_ref, b_ref, wp_ref, bp_ref, out_ref):
    i = pl.program_id(0)
    xv = x_ref[...]
    a8 = adj_ref[0].astype(jnp.float8_e4m3fn)
    deg = jnp.sum(a8.astype(jnp.float32), axis=1, keepdims=True)
    dis = jnp.where(deg > 0, jax.lax.rsqrt(deg), 0.0)
    y = (dis * xv).astype(jnp.float8_e4m3fn)
    z = jnp.dot(a8, y, preferred_element_type=jnp.float32)
    tx1 = (-(dis * z)).astype(jnp.bfloat16)
    xb = xv.astype(jnp.bfloat16)
    o = (lax.dot_general(xb, w0_ref[0].astype(jnp.bfloat16), _DN_T,
                         preferred_element_type=jnp.float32)
         + lax.dot_general(tx1, w1_ref[0].astype(jnp.bfloat16), _DN_T,
                           preferred_element_type=jnp.float32)
         + b_ref[0])
    contrib = lax.dot_general(o.astype(jnp.bfloat16),
                              wp_ref[...].astype(jnp.bfloat16), _DN_T,
                              preferred_element_type=jnp.float32)

    @pl.when(i == 0)
    def _init():
        out_ref[...] = contrib + bp_ref[...]

    @pl.when(i != 0)
    def _acc():
        out_ref[...] += contrib


def kernel(x, adj_list, W0, W1, b, Wp, bp):
    B, N, C = x.shape
    V = adj_list.shape[0]
    OUT = W0.shape[1]
    x2 = x.reshape(N, C)
    b3 = b.reshape(V, 1, OUT)
    bp2 = bp.reshape(1, OUT)

    out = pl.pallas_call(
        _body,
        grid=(V,),
        in_specs=[
            pl.BlockSpec((1, N, N), lambda i: (i, 0, 0)),
            pl.BlockSpec((N, C), lambda i: (0, 0)),
            pl.BlockSpec((1, OUT, C), lambda i: (i, 0, 0)),
            pl.BlockSpec((1, OUT, C), lambda i: (i, 0, 0)),
            pl.BlockSpec((1, 1, OUT), lambda i: (i, 0, 0)),
            pl.BlockSpec((OUT, OUT), lambda i: (0, i)),
            pl.BlockSpec((1, OUT), lambda i: (0, 0)),
        ],
        out_specs=pl.BlockSpec((N, OUT), lambda i: (0, 0)),
        out_shape=jax.ShapeDtypeStruct((N, OUT), jnp.float32),
        compiler_params=pltpu.CompilerParams(
            dimension_semantics=("arbitrary",),
        ),
    )(adj_list, x2, W0, W1, b3, Wp, bp2)
    return out.reshape(B, N, OUT)


# speedup vs baseline: 1.0518x; 1.0518x over previous
import jax
import jax.numpy as jnp
from jax import lax
from jax.experimental import pallas as pl
from jax.experimental.pallas import tpu as pltpu

_DN_T = (((1,), (1,)), ((), ()))


def _view(adj, xv, xb, w0, w1, bv, wp):
    deg = jnp.sum(adj, axis=1, keepdims=True)
    dis = jnp.where(deg > 0, jax.lax.rsqrt(deg), 0.0)
    y = (dis * xv).astype(jnp.float8_e4m3fn)
    z = jnp.dot(adj.astype(jnp.float8_e4m3fn), y,
                preferred_element_type=jnp.float32)
    tx1 = (-(dis * z)).astype(jnp.bfloat16)
    o = (lax.dot_general(xb, w0.astype(jnp.bfloat16), _DN_T,
                         preferred_element_type=jnp.float32)
         + lax.dot_general(tx1, w1.astype(jnp.bfloat16), _DN_T,
                           preferred_element_type=jnp.float32)
         + bv)
    return lax.dot_general(o.astype(jnp.bfloat16), wp.astype(jnp.bfloat16),
                           _DN_T, preferred_element_type=jnp.float32)


def _body(adj_hbm, x_ref, w0_ref, w1_ref, b_ref, wp_ref, bp_ref, out_ref,
          buf, sem):
    cp0 = pltpu.make_async_copy(adj_hbm.at[0], buf.at[0], sem.at[0])
    cp1 = pltpu.make_async_copy(adj_hbm.at[1], buf.at[1], sem.at[1])
    cp0.start()
    cp1.start()
    xv = x_ref[...]
    xb = xv.astype(jnp.bfloat16)
    OUT = out_ref.shape[1]
    cp0.wait()
    acc = (bp_ref[...]
           + _view(buf[0], xv, xb, w0_ref[0], w1_ref[0], b_ref[0],
                   wp_ref[:, 0:OUT]))
    cp1.wait()
    out_ref[...] = acc + _view(buf[1], xv, xb, w0_ref[1], w1_ref[1],
                               b_ref[1], wp_ref[:, OUT:2 * OUT])


def kernel(x, adj_list, W0, W1, b, Wp, bp):
    B, N, C = x.shape
    V = adj_list.shape[0]
    OUT = W0.shape[1]
    x2 = x.reshape(N, C)
    b3 = b.reshape(V, 1, OUT)
    bp2 = bp.reshape(1, OUT)

    out = pl.pallas_call(
        _body,
        in_specs=[
            pl.BlockSpec(memory_space=pltpu.MemorySpace.HBM),
            pl.BlockSpec((N, C), lambda: (0, 0)),
            pl.BlockSpec((V, OUT, C), lambda: (0, 0, 0)),
            pl.BlockSpec((V, OUT, C), lambda: (0, 0, 0)),
            pl.BlockSpec((V, 1, OUT), lambda: (0, 0, 0)),
            pl.BlockSpec((OUT, V * OUT), lambda: (0, 0)),
            pl.BlockSpec((1, OUT), lambda: (0, 0)),
        ],
        out_specs=pl.BlockSpec((N, OUT), lambda: (0, 0)),
        out_shape=jax.ShapeDtypeStruct((N, OUT), jnp.float32),
        scratch_shapes=[
            pltpu.VMEM((V, N, N), jnp.float32),
            pltpu.SemaphoreType.DMA((V,)),
        ],
    )(adj_list, x2, W0, W1, b3, Wp, bp2)
    return out.reshape(B, N, OUT)
